# Initial kernel scaffold; baseline (speedup 1.0000x reference)
import jax
import jax.numpy as jnp
from jax.experimental import pallas as pl


def kernel(x, W_group_weekday, W_group_time, W_group_gender, W_group_camp, W_group_grade, W_group_lane, W_group_district, W_group_area, W_group_r):
    raise NotImplementedError("write your pallas kernel here")


# speedup vs baseline: 1.6792x; 1.6792x over previous
import functools

import jax
import jax.numpy as jnp
from jax import lax
from jax.experimental import pallas as pl
from jax.experimental.pallas import tpu as pltpu
from jax.experimental.pallas import tpu_sc as plsc

_BOUNDS = ((0, 7), (7, 15), (15, 19), (19, 21), (21, 32),
           (32, 37), (37, 41), (41, 76), (76, 85))
_BATCH = 16384
_NCOL = 85
_D = 64
_NG = 9
_NW = 32
_ROWS_PER_W = _BATCH // _NW
_CHUNK = 128
_NCHUNK = _ROWS_PER_W // _CHUNK
_GATHER = _CHUNK * _NG
_L = 16


def _body(x_hbm, tab_hbm, out_hbm, x_v, idx_v, rows_v, sem):
  wid = lax.axis_index("s") * 2 + lax.axis_index("c")
  lane = lax.broadcasted_iota(jnp.int32, (_L,), 0)
  lane9 = lane * _NG

  def chunk_body(i, _):
    b0 = wid * _ROWS_PER_W + i * _CHUNK
    pltpu.sync_copy(x_hbm.at[pl.ds(b0 * _NCOL, _CHUNK * _NCOL)], x_v)

    def rowgrp_body(rg, _):
      rbase = (rg * _L + lane) * _NCOL
      for g, (s, e) in enumerate(_BOUNDS):
        cur = plsc.load_gather(x_v, [rbase + s])
        arg = jnp.full((_L,), s, jnp.int32)
        for c in range(s + 1, e):
          vals = plsc.load_gather(x_v, [rbase + c])
          m = vals > cur
          cur = jnp.where(m, vals, cur)
          arg = jnp.where(m, c, arg)
        pos = (rg * _L) * _NG + g + lane9
        plsc.store_scatter(idx_v, [pos], arg)
      return 0

    lax.fori_loop(0, _CHUNK // _L, rowgrp_body, 0)

    copies = []
    for k in range(_NG):
      copies.append(pltpu.async_copy(
          tab_hbm.at[idx_v.at[pl.ds(k * _CHUNK, _CHUNK)]],
          rows_v.at[pl.ds(k * _CHUNK, _CHUNK)], sem))
    for cp in copies:
      cp.wait()
    pltpu.sync_copy(rows_v, out_hbm.at[pl.ds(b0 * _NG, _GATHER)])
    return 0

  lax.fori_loop(0, _NCHUNK, chunk_body, 0)


def kernel(x, W_group_weekday, W_group_time, W_group_gender, W_group_camp,
           W_group_grade, W_group_lane, W_group_district, W_group_area,
           W_group_r):
  table = jnp.concatenate(
      (W_group_weekday, W_group_time, W_group_gender, W_group_camp,
       W_group_grade, W_group_lane, W_group_district, W_group_area,
       W_group_r), axis=0)
  mesh = plsc.VectorSubcoreMesh(core_axis_name="c", subcore_axis_name="s")
  f = pl.kernel(
      _body,
      mesh=mesh,
      compiler_params=pltpu.CompilerParams(
          needs_layout_passes=False, use_tc_tiling_on_sc=False),
      out_type=jax.ShapeDtypeStruct((_BATCH * _NG, _D), jnp.float32),
      scratch_types=[
          pltpu.VMEM((_CHUNK * _NCOL,), jnp.float32),
          pltpu.VMEM((_GATHER,), jnp.int32),
          pltpu.VMEM((_GATHER, _D), jnp.float32),
          pltpu.SemaphoreType.DMA,
      ],
  )
  out = f(x.reshape(_BATCH * _NCOL), table)
  return out.reshape(_BATCH, _NG * _D)
